# baseline (speedup 1.0000x reference)
import functools

import jax
import jax.numpy as jnp
from jax import lax
from jax.experimental import pallas as pl
from jax.experimental.pallas import tpu as pltpu
from jax.experimental.pallas import tpu_sc as plsc

BATCH = 4096
SEQ = 50
DIM = 64
B = BATCH * SEQ
NC = 2
NS = 16
NW = NC * NS
CPW = BATCH // NW
CH = CPW

_mesh = plsc.VectorSubcoreMesh(
    core_axis_name="c", subcore_axis_name="s", num_cores=NC, num_subcores=NS
)


@functools.partial(
    pl.kernel,
    out_type=jax.ShapeDtypeStruct((SEQ, 8, NW, 8, CH), jnp.float32),
    mesh=_mesh,
    scratch_types=[
        pltpu.VMEM((SEQ, CH), jnp.int32),
        pltpu.VMEM((CH, DIM), jnp.float32),
        pltpu.VMEM((CH, DIM), jnp.float32),
        pltpu.VMEM((8, 8, CH), jnp.float32),
        pltpu.VMEM((8, 8, CH), jnp.float32),
        pltpu.SemaphoreType.DMA,
        pltpu.SemaphoreType.DMA,
    ],
    compiler_params=pltpu.CompilerParams(
        use_tc_tiling_on_sc=False, needs_layout_passes=False
    ),
)
def _emb_lookup(
    idx_hbm, table_hbm, out_hbm, idx_v, rows_a, rows_b, tb_a, tb_b, gsem, ssem
):
    wid = lax.axis_index("s") * NC + lax.axis_index("c")
    col = wid * CPW
    pltpu.sync_copy(idx_hbm.at[:, pl.ds(col, CPW)], idx_v)

    iota = lax.iota(jnp.int32, 16)
    dtv = lax.shift_right_logical(iota, 3)
    rv = lax.bitwise_and(iota, 7)

    pltpu.async_copy(table_hbm.at[idx_v.at[0]], rows_a, gsem)
    pltpu.async_copy(table_hbm.at[idx_v.at[1]], rows_b, gsem)

    def step(j, rows_v, tb_v):
        pltpu.make_async_copy(
            table_hbm.at[idx_v.at[0]], rows_a, gsem
        ).wait()

        @pl.when(j >= 2)
        def _():
            pltpu.make_async_copy(
                tb_a, out_hbm.at[0, :, wid, :, :], ssem
            ).wait()

        @pl.loop(0, CH)
        def _(l):
            lf = jnp.zeros((16,), jnp.int32) + l
            for q in range(4):
                v = rows_v[l, pl.ds(16 * q, 16)]
                plsc.store_scatter(tb_v, [dtv + 2 * q, rv, lf], v)

        @pl.when(j + 2 < SEQ)
        def _():
            pltpu.async_copy(table_hbm.at[idx_v.at[j + 2]], rows_v, gsem)

        pltpu.async_copy(tb_v, out_hbm.at[j, :, wid, :, :], ssem)

    @pl.loop(0, SEQ)
    def _(j):
        even = lax.rem(j, 2) == 0

        @pl.when(even)
        def _():
            step(j, rows_a, tb_a)

        @pl.when(jnp.logical_not(even))
        def _():
            step(j, rows_b, tb_b)

    for _ in range(2):
        pltpu.make_async_copy(tb_a, out_hbm.at[0, :, wid, :, :], ssem).wait()


def kernel(token_ids, table):
    idx_t = token_ids.T.astype(jnp.int32)
    out5 = _emb_lookup(idx_t, table)
    return out5.transpose(2, 4, 0, 1, 3).reshape(BATCH, SEQ, DIM)
